# baseline (speedup 1.0000x reference)
import functools

import jax
import jax.numpy as jnp
from jax import lax
from jax.experimental import pallas as pl
from jax.experimental.pallas import tpu as pltpu
from jax.experimental.pallas import tpu_sc as plsc

NC, NS, L = 2, 16, 16
NW = NC * NS
IW = 128
G = 2
CHUNK = G * IW
NBUF = 2


def _build_sc_call(N, V, D, VROWS):
    n_per_w = N // NW
    n_chunks = n_per_w // CHUNK
    mesh = plsc.VectorSubcoreMesh(
        core_axis_name="c", subcore_axis_name="s",
        num_cores=NC, num_subcores=NS)

    @functools.partial(
        pl.kernel,
        out_type=jax.ShapeDtypeStruct((N, D), jnp.float32),
        mesh=mesh,
        scratch_types=[
            pltpu.HBM((VROWS, D), jnp.float32),
            pltpu.VMEM((VROWS, D), jnp.float32),
            pltpu.VMEM((D,), jnp.float32),
            [pltpu.VMEM((G, 2 * IW), jnp.int32) for _ in range(NBUF)],
            [pltpu.VMEM((G, IW), jnp.int32) for _ in range(NBUF)],
            [pltpu.VMEM((CHUNK, D), jnp.float32) for _ in range(NBUF)],
            [pltpu.SemaphoreType.DMA for _ in range(NBUF)],
            [pltpu.SemaphoreType.DMA for _ in range(NBUF)],
            pltpu.SemaphoreType.DMA,
        ],
    )
    def sc_fn(idm_hbm, table_hbm, bias_hbm, out_hbm,
              aug_hbm, aug_v, bias_v, in_v, eff_v, row_v, sem_i, sem_o,
              sem_g):
        cid = lax.axis_index("c")
        sid = lax.axis_index("s")
        wid = sid * NC + cid

        @pl.when(sid == 0)
        def _build():
            pltpu.sync_copy(table_hbm, aug_v.at[pl.ds(0, V)])
            pltpu.sync_copy(bias_hbm, bias_v)

            def add_bias(r, carry):
                for j in range(D // L):
                    sl = pl.ds(j * L, L)
                    aug_v[r, sl] = aug_v[r, sl] + bias_v[sl]
                return carry

            lax.fori_loop(0, V, add_bias, 0)
            zero = jnp.zeros((L,), jnp.float32)
            for r in range(V, VROWS):
                for j in range(D // L):
                    aug_v[r, pl.ds(j * L, L)] = zero
            pltpu.sync_copy(aug_v, aug_hbm)

        plsc.subcore_barrier()

        gbase = wid * (n_per_w // IW)

        def fire_input(c, b):
            pltpu.async_copy(idm_hbm.at[pl.ds(gbase + c * G, G)],
                             in_v[b], sem_i[b])

        def wait_input(c, b):
            pltpu.make_async_copy(idm_hbm.at[pl.ds(gbase + c * G, G)],
                                  in_v[b], sem_i[b]).wait()

        def wait_output(b):
            pltpu.make_async_copy(row_v[b], out_hbm.at[pl.ds(0, CHUNK)],
                                  sem_o[b]).wait()

        def process(c, b, fire_c2, wait_prev_out):
            if wait_prev_out:
                wait_output(b)
            wait_input(c, b)

            def sel(k, c2):
                sl = pl.ds(k * L, L)
                msl = pl.ds(k * L + IW, L)
                for g in range(G):
                    idv = in_v[b][g, sl]
                    mv = in_v[b][g, msl]
                    eff_v[b][g, sl] = jnp.where(mv != 0, V, idv)
                return c2

            lax.fori_loop(0, IW // L, sel, 0)

            if isinstance(fire_c2, bool):
                if fire_c2:
                    fire_input(c + NBUF, b)
            elif fire_c2 is not None:
                @pl.when(fire_c2)
                def _():
                    fire_input(c + NBUF, b)

            descs = [pltpu.async_copy(aug_hbm.at[eff_v[b].at[g]],
                                      row_v[b].at[pl.ds(g * IW, IW)],
                                      sem_g)
                     for g in range(G)]
            for d in descs:
                d.wait()
            pltpu.async_copy(row_v[b],
                             out_hbm.at[pl.ds((gbase + c * G) * IW, CHUNK)],
                             sem_o[b])

        for b in range(NBUF):
            fire_input(b, b)

        n_main = (n_chunks // NBUF) * NBUF

        def outer(o, carry):
            for b in range(NBUF):
                c = o * NBUF + b
                process(c, b,
                        fire_c2=(c + NBUF < n_chunks),
                        wait_prev_out=True)
            return carry

        for b in range(NBUF):
            process(b, b, fire_c2=bool(NBUF + b < n_chunks),
                    wait_prev_out=False)
        lax.fori_loop(1, n_main // NBUF, outer, 0)
        for c in range(n_main, n_chunks):
            process(c, c % NBUF, fire_c2=None, wait_prev_out=True)

        for b in range(NBUF):
            wait_output(b)

    return sc_fn


def kernel(phoneme_ids, padding_mask, table, pos_bias):
    B, T = phoneme_ids.shape
    V, D = table.shape
    N = B * T
    VROWS = ((V + 1 + 7) // 8) * 8

    ids = phoneme_ids.reshape(N // IW, IW).astype(jnp.int32)
    mask = padding_mask.reshape(N // IW, IW).astype(jnp.int32)
    idm = jnp.concatenate([ids, mask], axis=1)
    bias = pos_bias.reshape(D).astype(jnp.float32)

    sc_fn = _build_sc_call(N, V, D, VROWS)
    out = sc_fn(idm, table, bias)
    return out.reshape(B, T, D)


# speedup vs baseline: 8.2055x; 8.2055x over previous
import functools

import jax
import jax.numpy as jnp
from jax import lax
from jax.experimental import pallas as pl
from jax.experimental.pallas import tpu as pltpu
from jax.experimental.pallas import tpu_sc as plsc

NC, NS, L = 2, 16, 16
NW = NC * NS
IW = 128
G = 2
CHUNK = G * IW
NBUF = 2


def _build_sc_call(N, V, D, VROWS):
    n_per_w = N // NW
    n_chunks = n_per_w // CHUNK
    mesh = plsc.VectorSubcoreMesh(
        core_axis_name="c", subcore_axis_name="s",
        num_cores=NC, num_subcores=NS)

    @functools.partial(
        pl.kernel,
        out_type=jax.ShapeDtypeStruct((N, D), jnp.float32),
        mesh=mesh,
        scratch_types=[
            pltpu.VMEM_SHARED((VROWS, D), jnp.float32),
            pltpu.VMEM((VROWS, D), jnp.float32),
            pltpu.VMEM((D,), jnp.float32),
            [pltpu.VMEM((G, 2 * IW), jnp.int32) for _ in range(NBUF)],
            [pltpu.VMEM((G, IW), jnp.int32) for _ in range(NBUF)],
            [pltpu.VMEM((CHUNK, D), jnp.float32) for _ in range(NBUF)],
            [pltpu.SemaphoreType.DMA for _ in range(NBUF)],
            [pltpu.SemaphoreType.DMA for _ in range(NBUF)],
            pltpu.SemaphoreType.DMA,
        ],
    )
    def sc_fn(idm_hbm, table_hbm, bias_hbm, out_hbm,
              aug_sh, aug_v, bias_v, in_v, eff_v, row_v, sem_i, sem_o,
              sem_g):
        cid = lax.axis_index("c")
        sid = lax.axis_index("s")
        wid = sid * NC + cid

        @pl.when(sid == 0)
        def _build():
            pltpu.sync_copy(table_hbm, aug_v.at[pl.ds(0, V)])
            pltpu.sync_copy(bias_hbm, bias_v)

            def add_bias(r, carry):
                for j in range(D // L):
                    sl = pl.ds(j * L, L)
                    aug_v[r, sl] = aug_v[r, sl] + bias_v[sl]
                return carry

            lax.fori_loop(0, V, add_bias, 0)
            zero = jnp.zeros((L,), jnp.float32)
            for r in range(V, VROWS):
                for j in range(D // L):
                    aug_v[r, pl.ds(j * L, L)] = zero
            pltpu.sync_copy(aug_v, aug_sh)

        plsc.subcore_barrier()

        gbase = wid * (n_per_w // IW)

        def fire_input(c, b):
            pltpu.async_copy(idm_hbm.at[pl.ds(gbase + c * G, G)],
                             in_v[b], sem_i[b])

        def wait_input(c, b):
            pltpu.make_async_copy(idm_hbm.at[pl.ds(gbase + c * G, G)],
                                  in_v[b], sem_i[b]).wait()

        def wait_output(b):
            pltpu.make_async_copy(row_v[b], out_hbm.at[pl.ds(0, CHUNK)],
                                  sem_o[b]).wait()

        def process(c, b, fire_c2, wait_prev_out):
            if wait_prev_out:
                wait_output(b)
            wait_input(c, b)

            def sel(k, c2):
                sl = pl.ds(k * L, L)
                msl = pl.ds(k * L + IW, L)
                for g in range(G):
                    idv = in_v[b][g, sl]
                    mv = in_v[b][g, msl]
                    eff_v[b][g, sl] = jnp.where(mv != 0, V, idv)
                return c2

            lax.fori_loop(0, IW // L, sel, 0)

            if isinstance(fire_c2, bool):
                if fire_c2:
                    fire_input(c + NBUF, b)
            elif fire_c2 is not None:
                @pl.when(fire_c2)
                def _():
                    fire_input(c + NBUF, b)

            descs = [pltpu.async_copy(aug_sh.at[eff_v[b].at[g]],
                                      row_v[b].at[pl.ds(g * IW, IW)],
                                      sem_g)
                     for g in range(G)]
            for d in descs:
                d.wait()
            pltpu.async_copy(row_v[b],
                             out_hbm.at[pl.ds((gbase + c * G) * IW, CHUNK)],
                             sem_o[b])

        for b in range(NBUF):
            fire_input(b, b)

        n_main = (n_chunks // NBUF) * NBUF

        def outer(o, carry):
            for b in range(NBUF):
                c = o * NBUF + b
                process(c, b,
                        fire_c2=(c + NBUF < n_chunks),
                        wait_prev_out=True)
            return carry

        for b in range(NBUF):
            process(b, b, fire_c2=bool(NBUF + b < n_chunks),
                    wait_prev_out=False)
        lax.fori_loop(1, n_main // NBUF, outer, 0)
        for c in range(n_main, n_chunks):
            process(c, c % NBUF, fire_c2=None, wait_prev_out=True)

        for b in range(NBUF):
            wait_output(b)

    return sc_fn


def kernel(phoneme_ids, padding_mask, table, pos_bias):
    B, T = phoneme_ids.shape
    V, D = table.shape
    N = B * T
    VROWS = ((V + 1 + 7) // 8) * 8

    ids = phoneme_ids.reshape(N // IW, IW).astype(jnp.int32)
    mask = padding_mask.reshape(N // IW, IW).astype(jnp.int32)
    idm = jnp.concatenate([ids, mask], axis=1)
    bias = pos_bias.reshape(D).astype(jnp.float32)

    sc_fn = _build_sc_call(N, V, D, VROWS)
    out = sc_fn(idm, table, bias)
    return out.reshape(B, T, D)


# speedup vs baseline: 8.2200x; 1.0018x over previous
import functools

import jax
import jax.numpy as jnp
from jax import lax
from jax.experimental import pallas as pl
from jax.experimental.pallas import tpu as pltpu
from jax.experimental.pallas import tpu_sc as plsc

NC, NS, L = 2, 16, 16
NW = NC * NS
IW = 128
G = 2
CHUNK = G * IW
NBUF = 3


def _build_sc_call(N, V, D, VROWS):
    n_per_w = N // NW
    n_chunks = n_per_w // CHUNK
    rows_per_tile = VROWS // NS
    mesh = plsc.VectorSubcoreMesh(
        core_axis_name="c", subcore_axis_name="s",
        num_cores=NC, num_subcores=NS)

    @functools.partial(
        pl.kernel,
        out_type=jax.ShapeDtypeStruct((N, D), jnp.float32),
        mesh=mesh,
        scratch_types=[
            pltpu.VMEM_SHARED((VROWS, D), jnp.float32),
            pltpu.VMEM((1, D), jnp.float32),
            pltpu.VMEM((D,), jnp.float32),
            [pltpu.VMEM((G, 2 * IW), jnp.int32) for _ in range(NBUF)],
            [pltpu.VMEM((G, IW), jnp.int32) for _ in range(NBUF)],
            [pltpu.VMEM((CHUNK, D), jnp.float32) for _ in range(NBUF)],
            [pltpu.SemaphoreType.DMA for _ in range(NBUF)],
            [pltpu.SemaphoreType.DMA for _ in range(NBUF)],
            pltpu.SemaphoreType.DMA,
        ],
    )
    def sc_fn(idm_hbm, table_hbm, bias_hbm, out_hbm,
              aug_sh, row1_v, bias_v, in_v, eff_v, row_v, sem_i, sem_o,
              sem_g):
        cid = lax.axis_index("c")
        sid = lax.axis_index("s")
        wid = sid * NC + cid

        gbase = wid * (n_per_w // IW)

        def fire_input(c, b):
            pltpu.async_copy(idm_hbm.at[pl.ds(gbase + c * G, G)],
                             in_v[b], sem_i[b])

        def wait_input(c, b):
            pltpu.make_async_copy(idm_hbm.at[pl.ds(gbase + c * G, G)],
                                  in_v[b], sem_i[b]).wait()

        def wait_output(b):
            pltpu.make_async_copy(row_v[b], out_hbm.at[pl.ds(0, CHUNK)],
                                  sem_o[b]).wait()

        for b in range(NBUF):
            fire_input(b, b)

        pltpu.sync_copy(bias_hbm, bias_v)
        for k in range(rows_per_tile):
            r = sid * rows_per_tile + k

            @pl.when(r < V)
            def _table_row():
                pltpu.sync_copy(table_hbm.at[pl.ds(r, 1)], row1_v)
                for j in range(D // L):
                    sl = pl.ds(j * L, L)
                    row1_v[0, sl] = row1_v[0, sl] + bias_v[sl]
                pltpu.sync_copy(row1_v, aug_sh.at[pl.ds(r, 1)])

            @pl.when(r == V)
            def _zero_row():
                zero = jnp.zeros((L,), jnp.float32)
                for j in range(D // L):
                    row1_v[0, pl.ds(j * L, L)] = zero
                pltpu.sync_copy(row1_v, aug_sh.at[pl.ds(r, 1)])

        plsc.subcore_barrier()

        def process(c, b, fire_c2, wait_prev_out):
            if wait_prev_out:
                wait_output(b)
            wait_input(c, b)

            def sel(k, c2):
                sl = pl.ds(k * L, L)
                msl = pl.ds(k * L + IW, L)
                for g in range(G):
                    idv = in_v[b][g, sl]
                    mv = in_v[b][g, msl]
                    eff_v[b][g, sl] = jnp.where(mv != 0, V, idv)
                return c2

            lax.fori_loop(0, IW // L, sel, 0)

            if isinstance(fire_c2, bool):
                if fire_c2:
                    fire_input(c + NBUF, b)
            elif fire_c2 is not None:
                @pl.when(fire_c2)
                def _():
                    fire_input(c + NBUF, b)

            descs = [pltpu.async_copy(aug_sh.at[eff_v[b].at[g]],
                                      row_v[b].at[pl.ds(g * IW, IW)],
                                      sem_g)
                     for g in range(G)]
            for d in descs:
                d.wait()
            pltpu.async_copy(row_v[b],
                             out_hbm.at[pl.ds((gbase + c * G) * IW, CHUNK)],
                             sem_o[b])

        n_main = (n_chunks // NBUF) * NBUF

        for b in range(NBUF):
            process(b, b, fire_c2=bool(NBUF + b < n_chunks),
                    wait_prev_out=False)

        def outer(o, carry):
            for b in range(NBUF):
                c = o * NBUF + b
                process(c, b,
                        fire_c2=(c + NBUF < n_chunks),
                        wait_prev_out=True)
            return carry

        lax.fori_loop(1, n_main // NBUF, outer, 0)
        for c in range(n_main, n_chunks):
            process(c, c % NBUF, fire_c2=None, wait_prev_out=True)

        for b in range(NBUF):
            wait_output(b)

    return sc_fn


def kernel(phoneme_ids, padding_mask, table, pos_bias):
    B, T = phoneme_ids.shape
    V, D = table.shape
    N = B * T
    VROWS = ((V + 1 + NS - 1) // NS) * NS

    ids = phoneme_ids.reshape(N // IW, IW).astype(jnp.int32)
    mask = padding_mask.reshape(N // IW, IW).astype(jnp.int32)
    idm = jnp.concatenate([ids, mask], axis=1)
    bias = pos_bias.reshape(D).astype(jnp.float32)

    sc_fn = _build_sc_call(N, V, D, VROWS)
    out = sc_fn(idm, table, bias)
    return out.reshape(B, T, D)
